# baseline (speedup 1.0000x reference)
import jax
import jax.numpy as jnp
from jax.experimental import pallas as pl
from jax.experimental.pallas import tpu as pltpu
from jax.experimental.pallas import tpu_sc as plsc

_WINDOW = 512
_PCHUNK = 1024
_OCHUNK = 256


def _pack_body(xa_ref, xb_ref, o_ref):
    o_ref[...] = jnp.concatenate([xa_ref[...].T, xb_ref[...].T], axis=1)


def _pack_table(wt, npairs):
    h, v = wt.shape
    maxb = -(-v // _PCHUNK) - 1
    return pl.pallas_call(
        _pack_body,
        grid=(npairs,),
        in_specs=[
            pl.BlockSpec((h, _PCHUNK), lambda i: (0, 2 * i)),
            pl.BlockSpec(
                (h, _PCHUNK), lambda i: (0, jnp.minimum(2 * i + 1, maxb))
            ),
        ],
        out_specs=pl.BlockSpec((_PCHUNK, 2 * h), lambda i: (i, 0)),
        out_shape=jax.ShapeDtypeStruct((npairs * _PCHUNK, 2 * h), wt.dtype),
    )(wt, wt)


def _make_untranspose_body(h, half_b):
    def body(x_ref, o_ref):
        x = x_ref[...]
        o_ref[0, :, :half_b] = x[:, :h].T
        o_ref[0, :, half_b:] = x[:, h:].T

    return body


def _rows_to_out(rows2, ll, b, h):
    hb = b // 2
    return pl.pallas_call(
        _make_untranspose_body(h, hb),
        grid=(ll,),
        in_specs=[pl.BlockSpec((hb, 2 * h), lambda l: (l, 0))],
        out_specs=pl.BlockSpec((1, h, b), lambda l: (l, 0, 0)),
        out_shape=jax.ShapeDtypeStruct((ll, h, b), rows2.dtype),
    )(rows2)


def _sc_gather(table, idx, n, h):
    mesh = plsc.VectorSubcoreMesh(
        core_axis_name="core", subcore_axis_name="subcore"
    )

    @pl.kernel(
        out_type=jax.ShapeDtypeStruct((n, h), table.dtype),
        mesh=mesh,
        compiler_params=pltpu.CompilerParams(use_tc_tiling_on_sc=False),
    )
    def run(table_hbm, idx_hbm, out_hbm):
        def body(i_vmem, o_vmem):
            pltpu.sync_copy(table_hbm.at[i_vmem.at[0]], o_vmem)

        pltpu.emit_pipeline(
            body,
            grid=(n // _WINDOW,),
            in_specs=[
                pl.BlockSpec((1, _WINDOW), index_map=lambda i: (0, i))
            ],
            out_specs=[
                pl.BlockSpec((_WINDOW, h), index_map=lambda i: (i, 0))
            ],
            core_axis_name=("core", "subcore"),
            dimension_semantics=(pltpu.PARALLEL,),
        )(idx_hbm, out_hbm)

    return run(table, idx)


def kernel(inputs, weight):
    b, ll = inputs.shape
    v, h = weight.shape
    n = b * ll
    half_b = b // 2
    npairs = -(-v // (2 * _PCHUNK))
    vpad = npairs * 2 * _PCHUNK

    wt = weight.T
    idx_t = inputs.T.astype(jnp.int32)

    packed = _pack_table(wt, npairs)
    table_lin = packed.reshape(vpad, h)

    e = jnp.transpose(idx_t.reshape(ll, 2, half_b), (0, 2, 1)).reshape(1, n)
    be = e // _PCHUNK
    j = ((be // 2) * _PCHUNK + (e % _PCHUNK)) * 2 + (be % 2)

    rows = _sc_gather(table_lin, j, n, h)
    out_t = _rows_to_out(rows.reshape(n // 2, 2 * h), ll, b, h)
    return jnp.transpose(out_t, (2, 0, 1))
